# baseline (speedup 1.0000x reference)
import jax
import jax.numpy as jnp
from jax.experimental import pallas as pl
from jax.experimental.pallas import tpu as pltpu

N = 2048
D = 1024
H = 8
HD = D // H
E = 8
I = 2 * HD
TB = 256


def _mhmoe_block(x_ref, w_hpt_ref, b_hp_ref, embt_ref, w_up_ref,
                 w_down_ref, w_opt_ref, b_op_ref, y_ref):
    x = x_ref[...]
    h = jnp.dot(x, w_hpt_ref[...], preferred_element_type=jnp.float32)
    h = h + b_hp_ref[...]
    out = None
    for hh in range(H):
        hs = h[:, hh * HD:(hh + 1) * HD]
        logits = jnp.dot(hs, embt_ref[...],
                         preferred_element_type=jnp.float32)
        m = jnp.max(logits, axis=1, keepdims=True)
        ex = jnp.exp(logits - m)
        v = ex / jnp.sum(ex, axis=1, keepdims=True)
        iota = jax.lax.broadcasted_iota(jnp.int32, (TB, E), 1)
        m1 = jnp.max(v, axis=1, keepdims=True)
        i1 = jnp.min(jnp.where(v == m1, iota, E), axis=1, keepdims=True)
        sel1 = iota == i1
        vm = jnp.where(sel1, -1.0, v)
        m2 = jnp.max(vm, axis=1, keepdims=True)
        i2 = jnp.min(jnp.where(vm == m2, iota, E), axis=1, keepdims=True)
        sel2 = iota == i2
        w = jnp.where(sel1, m1, 0.0) + jnp.where(sel2, m2, 0.0)

        up = jnp.dot(hs, w_up_ref[...],
                     preferred_element_type=jnp.float32)
        a = jnp.square(jnp.maximum(up, 0.0))
        a = (a.reshape(TB, E, I) * w[:, :, None]).reshape(TB, E * I)
        dn = jnp.dot(a, w_down_ref[...],
                     preferred_element_type=jnp.float32)
        contrib = jnp.dot(dn, w_opt_ref[hh * HD:(hh + 1) * HD, :],
                          preferred_element_type=jnp.float32)
        out = contrib if out is None else out + contrib
    y_ref[...] = out + b_op_ref[...]


@jax.jit
def kernel(x, W_hp, b_hp, expert_emb, W_up, W_down, W_op, b_op):
    W_hpT = W_hp.T
    embT = expert_emb.T
    W_up_r = W_up.transpose(1, 0, 2).reshape(HD, E * I)
    W_down_r = W_down.reshape(E * I, HD)
    W_opT = W_op.T
    b_hp2 = b_hp.reshape(1, D)
    b_op2 = b_op.reshape(1, D)
    return pl.pallas_call(
        _mhmoe_block,
        grid=(N // TB,),
        in_specs=[
            pl.BlockSpec((TB, D), lambda i: (i, 0)),
            pl.BlockSpec((D, D), lambda i: (0, 0)),
            pl.BlockSpec((1, D), lambda i: (0, 0)),
            pl.BlockSpec((HD, E), lambda i: (0, 0)),
            pl.BlockSpec((HD, E * I), lambda i: (0, 0)),
            pl.BlockSpec((E * I, HD), lambda i: (0, 0)),
            pl.BlockSpec((D, D), lambda i: (0, 0)),
            pl.BlockSpec((1, D), lambda i: (0, 0)),
        ],
        out_specs=pl.BlockSpec((TB, D), lambda i: (i, 0)),
        out_shape=jax.ShapeDtypeStruct((N, D), jnp.float32),
        compiler_params=pltpu.CompilerParams(
            dimension_semantics=("parallel",)),
    )(x, W_hpT, b_hp2, embT, W_up_r, W_down_r, W_opT, b_op2)
